# baseline (speedup 1.0000x reference)
import functools

import jax
import jax.numpy as jnp
from jax import lax
from jax.experimental import pallas as pl
from jax.experimental.pallas import tpu as pltpu
from jax.experimental.pallas import tpu_sc as plsc

_NC = 2
_NS = 16
_W = _NC * _NS
_L = 16
_C = 80


def kernel(feat, edge_index):
    n_nodes, d = feat.shape
    e = edge_index.shape[1]
    per_w = e // _W
    n_chunks = per_w // _C
    assert per_w * _W == e and n_chunks * _C == per_w and d % (2 * _L) == 0
    assert n_chunks % 4 == 1
    dp = d // 2

    packed = jax.lax.bitcast_convert_type(
        feat.astype(jnp.bfloat16).reshape(n_nodes, dp, 2), jnp.float32)

    src = edge_index[0].reshape(_W, n_chunks, _C)
    dst = edge_index[1].reshape(_W, n_chunks, _C)

    mesh = plsc.VectorSubcoreMesh(
        core_axis_name="c", subcore_axis_name="s",
        num_cores=_NC, num_subcores=_NS)

    @functools.partial(
        pl.kernel,
        out_type=jax.ShapeDtypeStruct((_W, n_chunks, _C), jnp.float32),
        mesh=mesh,
        compiler_params=pltpu.CompilerParams(needs_layout_passes=False,
                                             use_tc_tiling_on_sc=False),
        scratch_types=[
            pltpu.VMEM((n_chunks, _C), jnp.int32),
            pltpu.VMEM((n_chunks, _C), jnp.int32),
            pltpu.VMEM((_C, dp), jnp.float32),
            pltpu.VMEM((_C, dp), jnp.float32),
            pltpu.VMEM((_C, dp), jnp.float32),
            pltpu.VMEM((_C, dp), jnp.float32),
            pltpu.VMEM((_C, dp), jnp.float32),
            pltpu.VMEM((_C, dp), jnp.float32),
            pltpu.VMEM((_C, dp), jnp.float32),
            pltpu.VMEM((_C, dp), jnp.float32),
            pltpu.VMEM((_C,), jnp.float32),
            pltpu.SemaphoreType.DMA,
            pltpu.SemaphoreType.DMA,
            pltpu.SemaphoreType.DMA,
            pltpu.SemaphoreType.DMA,
            pltpu.SemaphoreType.DMA,
            pltpu.SemaphoreType.DMA,
            pltpu.SemaphoreType.DMA,
            pltpu.SemaphoreType.DMA,
        ],
    )
    def ip_kernel(feat_h, src_h, dst_h, out_h, sidx, didx,
                  u0, v0, u1, v1, u2, v2, u3, v3, score,
                  su0, sv0, su1, sv1, su2, sv2, su3, sv3):
        cid = lax.axis_index("c")
        sid = lax.axis_index("s")
        wid = sid * _NC + cid

        pltpu.sync_copy(src_h.at[wid], sidx)
        pltpu.sync_copy(dst_h.at[wid], didx)

        lanes = lax.iota(jnp.int32, _L)
        bufs = ((u0, v0, su0, sv0), (u1, v1, su1, sv1),
                (u2, v2, su2, sv2), (u3, v3, su3, sv3))

        def issue(c, b):
            ub, vb, su, sv = bufs[b]
            pltpu.async_copy(feat_h.at[sidx.at[c]], ub, su)
            pltpu.async_copy(feat_h.at[didx.at[c]], vb, sv)

        def unpack2(x):
            return plsc.unpack(plsc.bitcast(x, jnp.bfloat16),
                               format=plsc.PackFormat.INTERLEAVED)

        def compute(c, b):
            ub, vb, su, sv = bufs[b]
            pltpu.make_async_copy(feat_h.at[sidx.at[c]], ub, su).wait()
            pltpu.make_async_copy(feat_h.at[didx.at[c]], vb, sv).wait()
            for g in range(_C // _L):
                tot = jnp.zeros((_L,), jnp.float32)
                for ee in range(_L):
                    row = g * _L + ee
                    acc = jnp.zeros((_L,), jnp.float32)
                    for k in range(dp // _L):
                        ua, ubb = unpack2(ub[row, pl.ds(k * _L, _L)])
                        va, vbb = unpack2(vb[row, pl.ds(k * _L, _L)])
                        acc = acc + ua * va + ubb * vbb
                    s = jnp.sum(acc)
                    tot = jnp.where(lanes == ee, s, tot)
                score[pl.ds(g * _L, _L)] = tot
            pltpu.sync_copy(score, out_h.at[wid, c])

        def issue_safe(c, b):
            @pl.when(c < n_chunks)
            def _():
                issue(c, b)

        issue(0, 0)
        issue(1, 1)
        issue(2, 2)

        def body(i, carry):
            c0 = 4 * i
            issue(c0 + 3, 3)
            compute(c0, 0)
            issue(c0 + 4, 0)
            compute(c0 + 1, 1)
            issue_safe(c0 + 5, 1)
            compute(c0 + 2, 2)
            issue_safe(c0 + 6, 2)
            compute(c0 + 3, 3)
            return c0 + 4

        last_c = lax.fori_loop(0, (n_chunks - 1) // 4, body, 0)
        compute(last_c, 0)

    out = ip_kernel(packed, src, dst)
    return out.reshape(e, 1)


# speedup vs baseline: 1.0629x; 1.0629x over previous
import functools

import jax
import jax.numpy as jnp
from jax import lax
from jax.experimental import pallas as pl
from jax.experimental.pallas import tpu as pltpu
from jax.experimental.pallas import tpu_sc as plsc

_NC = 2
_NS = 16
_W = _NC * _NS
_L = 16
_C = 80


def kernel(feat, edge_index):
    n_nodes, d = feat.shape
    e = edge_index.shape[1]
    per_w = e // _W
    n_chunks = per_w // _C
    assert per_w * _W == e and n_chunks * _C == per_w and d % (2 * _L) == 0
    assert n_chunks % 2 == 1
    dp = d // 2

    packed = jax.lax.bitcast_convert_type(
        feat.astype(jnp.bfloat16).reshape(n_nodes, dp, 2), jnp.float32)

    src = edge_index[0].reshape(_W, n_chunks, _C)
    dst = edge_index[1].reshape(_W, n_chunks, _C)

    mesh = plsc.VectorSubcoreMesh(
        core_axis_name="c", subcore_axis_name="s",
        num_cores=_NC, num_subcores=_NS)

    @functools.partial(
        pl.kernel,
        out_type=jax.ShapeDtypeStruct((_W, n_chunks, _C), jnp.float32),
        mesh=mesh,
        compiler_params=pltpu.CompilerParams(needs_layout_passes=False,
                                             use_tc_tiling_on_sc=False),
        scratch_types=[
            pltpu.VMEM((n_chunks, _C), jnp.int32),
            pltpu.VMEM((n_chunks, _C), jnp.int32),
            pltpu.VMEM((_C, dp), jnp.float32),
            pltpu.VMEM((_C, dp), jnp.float32),
            pltpu.VMEM((_C, dp), jnp.float32),
            pltpu.VMEM((_C, dp), jnp.float32),
            pltpu.VMEM((_C,), jnp.float32),
            pltpu.SemaphoreType.DMA,
            pltpu.SemaphoreType.DMA,
            pltpu.SemaphoreType.DMA,
            pltpu.SemaphoreType.DMA,
        ],
    )
    def ip_kernel(feat_h, src_h, dst_h, out_h, sidx, didx,
                  u0, v0, u1, v1, score, su0, sv0, su1, sv1):
        cid = lax.axis_index("c")
        sid = lax.axis_index("s")
        wid = sid * _NC + cid

        pltpu.sync_copy(src_h.at[wid], sidx)
        pltpu.sync_copy(dst_h.at[wid], didx)

        lanes = lax.iota(jnp.int32, _L)
        bufs = ((u0, v0, su0, sv0), (u1, v1, su1, sv1))

        def issue(c, b):
            ub, vb, su, sv = bufs[b]
            pltpu.async_copy(feat_h.at[sidx.at[c]], ub, su)
            pltpu.async_copy(feat_h.at[didx.at[c]], vb, sv)

        def unpack2(x):
            return plsc.unpack(plsc.bitcast(x, jnp.bfloat16),
                               format=plsc.PackFormat.INTERLEAVED)

        nk = dp // _L

        def compute(c, b):
            ub, vb, su, sv = bufs[b]
            pltpu.make_async_copy(feat_h.at[sidx.at[c]], ub, su).wait()
            pltpu.make_async_copy(feat_h.at[didx.at[c]], vb, sv).wait()

            @plsc.parallel_loop(0, _C // _L, unroll=_C // _L)
            def _group(g):
                base = pl.multiple_of(g * _L, _L)
                tot = jnp.zeros((_L,), jnp.float32)
                for ee in range(_L):
                    acc0 = jnp.zeros((_L,), jnp.float32)
                    acc1 = jnp.zeros((_L,), jnp.float32)
                    for k in range(nk):
                        ua, ubb = unpack2(ub[base + ee, pl.ds(k * _L, _L)])
                        va, vbb = unpack2(vb[base + ee, pl.ds(k * _L, _L)])
                        acc0 = acc0 + ua * va
                        acc1 = acc1 + ubb * vbb
                    s = jnp.sum(acc0 + acc1)
                    tot = jnp.where(lanes == ee, s, tot)
                score[pl.ds(base, _L)] = tot

            pltpu.sync_copy(score, out_h.at[wid, c])

        issue(0, 0)

        def body(i, carry):
            c0 = 2 * i
            issue(c0 + 1, 1)
            compute(c0, 0)
            issue(c0 + 2, 0)
            compute(c0 + 1, 1)
            return c0 + 2

        last_c = lax.fori_loop(0, (n_chunks - 1) // 2, body, 0)
        compute(last_c, 0)

    out = ip_kernel(packed, src, dst)
    return out.reshape(e, 1)


# speedup vs baseline: 1.1169x; 1.0508x over previous
import functools

import jax
import jax.numpy as jnp
from jax import lax
from jax.experimental import pallas as pl
from jax.experimental.pallas import tpu as pltpu
from jax.experimental.pallas import tpu_sc as plsc

_NC = 2
_NS = 16
_W = _NC * _NS
_L = 16
_C = 80


def kernel(feat, edge_index):
    n_nodes, d = feat.shape
    e = edge_index.shape[1]
    per_w = e // _W
    n_chunks = per_w // _C
    assert per_w * _W == e and n_chunks * _C == per_w and d % (2 * _L) == 0
    assert n_chunks % 2 == 1
    dp = d // 2

    packed = jax.lax.bitcast_convert_type(
        feat.astype(jnp.bfloat16).reshape(n_nodes, dp, 2), jnp.float32)

    src = edge_index[0].reshape(_W, n_chunks, _C)
    dst = edge_index[1].reshape(_W, n_chunks, _C)

    mesh = plsc.VectorSubcoreMesh(
        core_axis_name="c", subcore_axis_name="s",
        num_cores=_NC, num_subcores=_NS)

    @functools.partial(
        pl.kernel,
        out_type=jax.ShapeDtypeStruct((_W, n_chunks, _C), jnp.float32),
        mesh=mesh,
        compiler_params=pltpu.CompilerParams(needs_layout_passes=False,
                                             use_tc_tiling_on_sc=False),
        scratch_types=[
            pltpu.VMEM((n_chunks, _C), jnp.int32),
            pltpu.VMEM((n_chunks, _C), jnp.int32),
            pltpu.VMEM((_C, dp), jnp.float32),
            pltpu.VMEM((_C, dp), jnp.float32),
            pltpu.VMEM((_C, dp), jnp.float32),
            pltpu.VMEM((_C, dp), jnp.float32),
            pltpu.VMEM((_C,), jnp.float32),
            pltpu.VMEM_SHARED((n_nodes, dp), jnp.float32),
            pltpu.SemaphoreType.DMA,
            pltpu.SemaphoreType.DMA,
            pltpu.SemaphoreType.DMA,
            pltpu.SemaphoreType.DMA,
        ],
    )
    def ip_kernel(feat_h, src_h, dst_h, out_h, sidx, didx,
                  u0, v0, u1, v1, score, shared, su0, sv0, su1, sv1):
        cid = lax.axis_index("c")
        sid = lax.axis_index("s")
        wid = sid * _NC + cid

        @pl.when(sid == 0)
        def _():
            pltpu.sync_copy(feat_h, shared)

        pltpu.sync_copy(src_h.at[wid], sidx)
        pltpu.sync_copy(dst_h.at[wid], didx)
        plsc.subcore_barrier()

        lanes = lax.iota(jnp.int32, _L)
        bufs = ((u0, v0, su0, sv0), (u1, v1, su1, sv1))

        def issue(c, b):
            ub, vb, su, sv = bufs[b]
            pltpu.async_copy(feat_h.at[sidx.at[c]], ub, su)
            pltpu.async_copy(shared.at[didx.at[c]], vb, sv)

        def unpack2(x):
            return plsc.unpack(plsc.bitcast(x, jnp.bfloat16),
                               format=plsc.PackFormat.INTERLEAVED)

        def compute(c, b):
            ub, vb, su, sv = bufs[b]
            pltpu.make_async_copy(feat_h.at[sidx.at[c]], ub, su).wait()
            pltpu.make_async_copy(shared.at[didx.at[c]], vb, sv).wait()
            for g in range(_C // _L):
                tot = jnp.zeros((_L,), jnp.float32)
                for ee in range(_L):
                    row = g * _L + ee
                    acc = jnp.zeros((_L,), jnp.float32)
                    for k in range(dp // _L):
                        ua, ubb = unpack2(ub[row, pl.ds(k * _L, _L)])
                        va, vbb = unpack2(vb[row, pl.ds(k * _L, _L)])
                        acc = acc + ua * va + ubb * vbb
                    s = jnp.sum(acc)
                    tot = jnp.where(lanes == ee, s, tot)
                score[pl.ds(g * _L, _L)] = tot
            pltpu.sync_copy(score, out_h.at[wid, c])

        issue(0, 0)

        def body(i, carry):
            c0 = 2 * i
            issue(c0 + 1, 1)
            compute(c0, 0)
            issue(c0 + 2, 0)
            compute(c0 + 1, 1)
            return c0 + 2

        last_c = lax.fori_loop(0, (n_chunks - 1) // 2, body, 0)
        compute(last_c, 0)

    out = ip_kernel(packed, src, dst)
    return out.reshape(e, 1)


# speedup vs baseline: 1.2353x; 1.1060x over previous
import functools

import jax
import jax.numpy as jnp
from jax import lax
from jax.experimental import pallas as pl
from jax.experimental.pallas import tpu as pltpu
from jax.experimental.pallas import tpu_sc as plsc

_NC = 2
_NS = 16
_W = _NC * _NS
_L = 16
_C = 80


def kernel(feat, edge_index):
    n_nodes, d = feat.shape
    e = edge_index.shape[1]
    per_w = e // _W
    n_chunks = per_w // _C
    assert per_w * _W == e and n_chunks * _C == per_w and d % (2 * _L) == 0
    assert n_chunks % 2 == 1

    packed = feat.astype(jnp.bfloat16)

    src = edge_index[0].reshape(_W, n_chunks, _C)
    dst = edge_index[1].reshape(_W, n_chunks, _C)

    mesh = plsc.VectorSubcoreMesh(
        core_axis_name="c", subcore_axis_name="s",
        num_cores=_NC, num_subcores=_NS)

    @functools.partial(
        pl.kernel,
        out_type=jax.ShapeDtypeStruct((_W, n_chunks, _C), jnp.float32),
        mesh=mesh,
        compiler_params=pltpu.CompilerParams(needs_layout_passes=False,
                                             use_tc_tiling_on_sc=False),
        scratch_types=[
            pltpu.VMEM((n_chunks, _C), jnp.int32),
            pltpu.VMEM((n_chunks, _C), jnp.int32),
            pltpu.VMEM((_C, d), jnp.bfloat16),
            pltpu.VMEM((_C, d), jnp.bfloat16),
            pltpu.VMEM((_C, d), jnp.bfloat16),
            pltpu.VMEM((_C, d), jnp.bfloat16),
            pltpu.VMEM((_C,), jnp.float32),
            pltpu.VMEM_SHARED((n_nodes, d), jnp.bfloat16),
            pltpu.SemaphoreType.DMA,
            pltpu.SemaphoreType.DMA,
            pltpu.SemaphoreType.DMA,
            pltpu.SemaphoreType.DMA,
        ],
    )
    def ip_kernel(feat_h, src_h, dst_h, out_h, sidx, didx,
                  u0, v0, u1, v1, score, shared, su0, sv0, su1, sv1):
        cid = lax.axis_index("c")
        sid = lax.axis_index("s")
        wid = sid * _NC + cid

        @pl.when(sid == 0)
        def _():
            pltpu.sync_copy(feat_h, shared)

        pltpu.sync_copy(src_h.at[wid], sidx)
        pltpu.sync_copy(dst_h.at[wid], didx)
        plsc.subcore_barrier()

        lanes = lax.iota(jnp.int32, _L)
        bufs = ((u0, v0, su0, sv0), (u1, v1, su1, sv1))

        def issue(c, b):
            ub, vb, su, sv = bufs[b]
            pltpu.async_copy(feat_h.at[sidx.at[c]], ub, su)
            pltpu.async_copy(shared.at[didx.at[c]], vb, sv)

        def unpack2(x):
            return plsc.unpack(x, format=plsc.PackFormat.INTERLEAVED)

        def compute(c, b):
            ub, vb, su, sv = bufs[b]
            pltpu.make_async_copy(feat_h.at[sidx.at[c]], ub, su).wait()
            pltpu.make_async_copy(shared.at[didx.at[c]], vb, sv).wait()
            for g in range(_C // _L):
                tot = jnp.zeros((_L,), jnp.float32)
                for ee in range(_L):
                    row = g * _L + ee
                    acc = jnp.zeros((_L,), jnp.float32)
                    for k in range(d // (2 * _L)):
                        ua, ubb = unpack2(ub[row, pl.ds(k * 2 * _L, 2 * _L)])
                        va, vbb = unpack2(vb[row, pl.ds(k * 2 * _L, 2 * _L)])
                        acc = acc + ua * va + ubb * vbb
                    s = jnp.sum(acc)
                    tot = jnp.where(lanes == ee, s, tot)
                score[pl.ds(g * _L, _L)] = tot
            pltpu.sync_copy(score, out_h.at[wid, c])

        issue(0, 0)

        def body(i, carry):
            c0 = 2 * i
            issue(c0 + 1, 1)
            compute(c0, 0)
            issue(c0 + 2, 0)
            compute(c0 + 1, 1)
            return c0 + 2

        last_c = lax.fori_loop(0, (n_chunks - 1) // 2, body, 0)
        compute(last_c, 0)

    out = ip_kernel(packed, src, dst)
    return out.reshape(e, 1)


# speedup vs baseline: 1.2869x; 1.0418x over previous
import functools

import jax
import jax.numpy as jnp
from jax import lax
from jax.experimental import pallas as pl
from jax.experimental.pallas import tpu as pltpu
from jax.experimental.pallas import tpu_sc as plsc

_NC = 2
_NS = 16
_W = _NC * _NS
_L = 16
_C = 80


def kernel(feat, edge_index):
    n_nodes, d = feat.shape
    e = edge_index.shape[1]
    per_w = e // _W
    n_chunks = per_w // _C
    assert per_w * _W == e and n_chunks * _C == per_w and d % (2 * _L) == 0
    assert n_chunks % 2 == 1

    packed = feat.astype(jnp.bfloat16)

    edges = edge_index.reshape(2 * _W, n_chunks, _C)

    mesh = plsc.VectorSubcoreMesh(
        core_axis_name="c", subcore_axis_name="s",
        num_cores=_NC, num_subcores=_NS)

    @functools.partial(
        pl.kernel,
        out_type=jax.ShapeDtypeStruct((e,), jnp.float32),
        mesh=mesh,
        compiler_params=pltpu.CompilerParams(needs_layout_passes=False,
                                             use_tc_tiling_on_sc=False),
        scratch_types=[
            pltpu.VMEM((n_chunks, _C), jnp.int32),
            pltpu.VMEM((n_chunks, _C), jnp.int32),
            pltpu.VMEM((_C, d), jnp.bfloat16),
            pltpu.VMEM((_C, d), jnp.bfloat16),
            pltpu.VMEM((_C, d), jnp.bfloat16),
            pltpu.VMEM((_C, d), jnp.bfloat16),
            pltpu.VMEM((_C,), jnp.float32),
            pltpu.VMEM_SHARED((n_nodes, d), jnp.bfloat16),
            pltpu.SemaphoreType.DMA,
            pltpu.SemaphoreType.DMA,
            pltpu.SemaphoreType.DMA,
            pltpu.SemaphoreType.DMA,
        ],
    )
    def ip_kernel(feat_h, edges_h, out_h, sidx, didx,
                  u0, v0, u1, v1, score, shared, su0, sv0, su1, sv1):
        cid = lax.axis_index("c")
        sid = lax.axis_index("s")
        wid = sid * _NC + cid

        @pl.when(sid == 0)
        def _():
            pltpu.sync_copy(feat_h, shared)

        pltpu.sync_copy(edges_h.at[wid], sidx)
        pltpu.sync_copy(edges_h.at[_W + wid], didx)
        plsc.subcore_barrier()

        lanes = lax.iota(jnp.int32, _L)
        bufs = ((u0, v0, su0, sv0), (u1, v1, su1, sv1))

        def issue(c, b):
            ub, vb, su, sv = bufs[b]
            pltpu.async_copy(feat_h.at[sidx.at[c]], ub, su)
            pltpu.async_copy(shared.at[didx.at[c]], vb, sv)

        def unpack2(x):
            return plsc.unpack(x, format=plsc.PackFormat.INTERLEAVED)

        def compute(c, b):
            ub, vb, su, sv = bufs[b]
            pltpu.make_async_copy(feat_h.at[sidx.at[c]], ub, su).wait()
            pltpu.make_async_copy(shared.at[didx.at[c]], vb, sv).wait()
            for g in range(_C // _L):
                tot = jnp.zeros((_L,), jnp.float32)
                for ee in range(_L):
                    row = g * _L + ee
                    acc = jnp.zeros((_L,), jnp.float32)
                    for k in range(d // (2 * _L)):
                        ua, ubb = unpack2(ub[row, pl.ds(k * 2 * _L, 2 * _L)])
                        va, vbb = unpack2(vb[row, pl.ds(k * 2 * _L, 2 * _L)])
                        acc = acc + ua * va + ubb * vbb
                    s = jnp.sum(acc)
                    tot = jnp.where(lanes == ee, s, tot)
                score[pl.ds(g * _L, _L)] = tot
            pltpu.sync_copy(score, out_h.at[pl.ds(wid * per_w + c * _C, _C)])

        issue(0, 0)

        def body(i, carry):
            c0 = 2 * i
            issue(c0 + 1, 1)
            compute(c0, 0)
            issue(c0 + 2, 0)
            compute(c0 + 1, 1)
            return c0 + 2

        last_c = lax.fori_loop(0, (n_chunks - 1) // 2, body, 0)
        compute(last_c, 0)

    out = ip_kernel(packed, edges)
    return out.reshape(e, 1)
